# baseline (speedup 1.0000x reference)
import functools

import jax
import jax.numpy as jnp
from jax import lax
from jax.experimental import pallas as pl
from jax.experimental.pallas import tpu as pltpu
from jax.experimental.pallas import tpu_sc as plsc

B = 16
G = 64
S = 16
D = 128
N_GENES = 20000

NC = 2
NS = 16
NW = NC * NS
LANES = 16

SC_B = 4
TC_B = B - SC_B

GROUPS_PER_W = (SC_B * G) // NW
ROWS_PER_W = GROUPS_PER_W * S
PARTS = G // GROUPS_PER_W


HROWS = ROWS_PER_W // 2
HGROUPS = GROUPS_PER_W // 2


def _sc_body(x_hbm, out_hbm, in_v, out_v, sem0, sem1):
    wid = lax.axis_index("s") * NC + lax.axis_index("c")
    b = wid // PARTS
    part = wid % PARTS
    in_base = b * N_GENES + part * ROWS_PER_W
    out_base = wid * GROUPS_PER_W

    copies = [
        pltpu.async_copy(
            x_hbm.at[pl.ds(in_base + h * HROWS, HROWS), :],
            in_v.at[pl.ds(h * HROWS, HROWS), :], sem)
        for h, sem in enumerate((sem0, sem1))
    ]

    for h in range(2):
        copies[h].wait()

        @plsc.parallel_loop(h * HGROUPS, (h + 1) * HGROUPS, unroll=2)
        def gbody(g):
            row0 = g * S
            for dc in range(D // LANES):
                sl = pl.ds(dc * LANES, LANES)
                vals = [in_v[row0 + s, sl] for s in range(S)]
                while len(vals) > 1:
                    vals = [vals[i] + vals[i + 1]
                            for i in range(0, len(vals), 2)]
                out_v[g, sl] = vals[0] * (1.0 / S)

    pltpu.sync_copy(out_v, out_hbm.at[pl.ds(out_base, GROUPS_PER_W), :])


_sc_kernel = functools.partial(
    pl.kernel,
    out_type=jax.ShapeDtypeStruct((SC_B * G, D), jnp.float32),
    mesh=plsc.VectorSubcoreMesh(core_axis_name="c", subcore_axis_name="s"),
    scratch_types=[
        pltpu.VMEM((ROWS_PER_W, D), jnp.float32),
        pltpu.VMEM((GROUPS_PER_W, D), jnp.float32),
        pltpu.SemaphoreType.DMA,
        pltpu.SemaphoreType.DMA,
    ],
)(_sc_body)


TC_BLK = 4


def _tc_body(x_ref, o_ref):
    x = x_ref[...]
    o_ref[...] = x.reshape(TC_BLK, G, S, D).sum(axis=2) * (1.0 / S)


_tc_kernel = pl.pallas_call(
    _tc_body,
    grid=(TC_B // TC_BLK,),
    in_specs=[pl.BlockSpec((TC_BLK, G * S, D),
                           lambda i: (SC_B // TC_BLK + i, 0, 0))],
    out_specs=pl.BlockSpec((TC_BLK, G, D),
                           lambda i: (SC_B // TC_BLK + i, 0, 0)),
    out_shape=jax.ShapeDtypeStruct((B, G, D), jnp.float32),
)


@jax.jit
def kernel(gene_output):
    flat = gene_output.reshape(B * N_GENES, D)
    sc_out = _sc_kernel(flat).reshape(SC_B, G, D)
    out_full = _tc_kernel(gene_output)
    return lax.dynamic_update_slice(out_full, sc_out, (0, 0, 0))


# speedup vs baseline: 1.0312x; 1.0312x over previous
import functools

import jax
import jax.numpy as jnp
from jax import lax
from jax.experimental import pallas as pl
from jax.experimental.pallas import tpu as pltpu
from jax.experimental.pallas import tpu_sc as plsc

B = 16
G = 64
S = 16
D = 128
N_GENES = 20000

NC = 2
NS = 16
NW = NC * NS
LANES = 16

SC_B = 4
TC_B = B - SC_B

GROUPS_PER_W = (SC_B * G) // NW
ROWS_PER_W = GROUPS_PER_W * S
PARTS = G // GROUPS_PER_W


def _sc_body(x_hbm, out_hbm, in_v, out_v):
    wid = lax.axis_index("s") * NC + lax.axis_index("c")
    b = wid // PARTS
    part = wid % PARTS
    in_base = b * N_GENES + part * ROWS_PER_W
    out_base = wid * GROUPS_PER_W

    pltpu.sync_copy(x_hbm.at[pl.ds(in_base, ROWS_PER_W), :], in_v)

    @plsc.parallel_loop(0, GROUPS_PER_W, unroll=2)
    def gbody(g):
        row0 = g * S
        for dc in range(D // LANES):
            sl = pl.ds(dc * LANES, LANES)
            vals = [in_v[row0 + s, sl] for s in range(S)]
            while len(vals) > 1:
                vals = [vals[i] + vals[i + 1] for i in range(0, len(vals), 2)]
            out_v[g, sl] = vals[0] * (1.0 / S)

    pltpu.sync_copy(out_v, out_hbm.at[pl.ds(out_base, GROUPS_PER_W), :])


_sc_kernel = functools.partial(
    pl.kernel,
    out_type=jax.ShapeDtypeStruct((SC_B * G, D), jnp.float32),
    mesh=plsc.VectorSubcoreMesh(core_axis_name="c", subcore_axis_name="s"),
    scratch_types=[
        pltpu.VMEM((ROWS_PER_W, D), jnp.float32),
        pltpu.VMEM((GROUPS_PER_W, D), jnp.float32),
    ],
)(_sc_body)


TC_BLK = 4


def _tc_body(x_ref, o_ref):
    x = x_ref[...]
    o_ref[...] = x.reshape(TC_BLK, G, S, D).sum(axis=2) * (1.0 / S)


_tc_kernel = pl.pallas_call(
    _tc_body,
    grid=(TC_B // TC_BLK,),
    in_specs=[pl.BlockSpec((TC_BLK, G * S, D),
                           lambda i: (SC_B // TC_BLK + i, 0, 0))],
    out_specs=pl.BlockSpec((TC_BLK, G, D),
                           lambda i: (SC_B // TC_BLK + i, 0, 0)),
    out_shape=jax.ShapeDtypeStruct((B, G, D), jnp.float32),
)


@jax.jit
def kernel(gene_output):
    flat = gene_output.reshape(B * N_GENES, D)
    sc_out = _sc_kernel(flat).reshape(SC_B, G, D)
    out_full = _tc_kernel(gene_output)
    return lax.dynamic_update_slice(out_full, sc_out, (0, 0, 0))
